# Initial kernel scaffold; baseline (speedup 1.0000x reference)
import jax
import jax.numpy as jnp
from jax.experimental import pallas as pl


def kernel(audio, params):
    raise NotImplementedError("write your pallas kernel here")


# speedup vs baseline: 1.0065x; 1.0065x over previous
import functools

import jax
import jax.numpy as jnp
import numpy as np
from jax.experimental import pallas as pl
from jax.experimental.pallas import tpu as pltpu

ENC_DIM = 512
N_CODEBOOKS = 8
CODEBOOK_SIZE = 2048
N_HEADS = 8
ENC_STRIDES = [4, 8, 5, 4, 3]
DEC_STRIDES = [3, 4, 5, 8, 4]
MAX_SEQ_LEN = 4096


def _rvq_stage(z_ref, cb_ref, cbt_ref, cbsq_ref, out_ref, res_ref, acc_ref):
    qi = pl.program_id(0)
    cb = cb_ref[0]
    cbt = cbt_ref[0]
    B, T, D = z_ref.shape
    N = B * T

    @pl.when(qi == 0)
    def _init():
        res_ref[...] = z_ref[...].reshape(N, D)
        acc_ref[...] = jnp.zeros_like(acc_ref)

    res = res_ref[...]
    mm = jax.lax.dot_general(
        res, cbt, (((1,), (0,)), ((), ())),
        preferred_element_type=jnp.float32)
    d = cbsq_ref[0][0][None, :] - 2.0 * mm
    dmin = jnp.min(d, axis=-1, keepdims=True)
    iota = jax.lax.broadcasted_iota(jnp.int32, d.shape, 1)
    idx = jnp.min(jnp.where(d <= dmin, iota, CODEBOOK_SIZE), axis=-1,
                  keepdims=True)
    onehot = (iota == idx).astype(jnp.float32)
    sel = jax.lax.dot_general(
        onehot, cb, (((1,), (0,)), ((), ())),
        preferred_element_type=jnp.float32,
        precision=jax.lax.Precision.HIGHEST)
    acc_ref[...] += sel
    res_ref[...] = res - sel

    @pl.when(qi == N_CODEBOOKS - 1)
    def _final():
        z = z_ref[...].reshape(N, D)
        out_ref[...] = (z + (acc_ref[...] - z)).reshape(B, T, D)


@jax.jit
def _rvq_quantize(z, codebooks):
    B, T, D = z.shape
    N = B * T
    cbt = codebooks.transpose(0, 2, 1)
    cbsq = jnp.sum(codebooks * codebooks, axis=-1)[:, None, :]
    return pl.pallas_call(
        _rvq_stage,
        grid=(N_CODEBOOKS,),
        in_specs=[
            pl.BlockSpec((B, T, D), lambda q: (0, 0, 0)),
            pl.BlockSpec((1, CODEBOOK_SIZE, D), lambda q: (q, 0, 0)),
            pl.BlockSpec((1, D, CODEBOOK_SIZE), lambda q: (q, 0, 0)),
            pl.BlockSpec((1, 1, CODEBOOK_SIZE), lambda q: (q, 0, 0)),
        ],
        out_specs=pl.BlockSpec((B, T, D), lambda q: (0, 0, 0)),
        out_shape=jax.ShapeDtypeStruct((B, T, D), jnp.float32),
        scratch_shapes=[pltpu.VMEM((N, D), jnp.float32),
                        pltpu.VMEM((N, D), jnp.float32)],
    )(z, codebooks, cbt, cbsq)


def _conv1d(x, w, b, stride=1, padding=0, dilation=1):
    out = jax.lax.conv_general_dilated(
        x, w, window_strides=(stride,), padding=[(padding, padding)],
        rhs_dilation=(dilation,), dimension_numbers=('NCH', 'OIH', 'NCH'))
    return out + b[None, :, None]


def _conv_transpose1d(x, w, b, stride, padding):
    k = w.shape[2]
    w_conv = jnp.flip(w, axis=2).transpose(1, 0, 2)
    pad = k - 1 - padding
    out = jax.lax.conv_general_dilated(
        x, w_conv, window_strides=(1,), padding=[(pad, pad)],
        lhs_dilation=(stride,), dimension_numbers=('NCH', 'OIH', 'NCH'))
    return out + b[None, :, None]


def _silu(x):
    return x * jax.nn.sigmoid(x)


def _layer_norm(x, g, b, eps=1e-5):
    mu = jnp.mean(x, axis=-1, keepdims=True)
    var = jnp.var(x, axis=-1, keepdims=True)
    return (x - mu) / jnp.sqrt(var + eps) * g + b


def _rope_tables(dim, max_len, theta=10000.0):
    freqs = 1.0 / theta ** (jnp.arange(0, dim, 2, dtype=jnp.float32) / dim)
    t = jnp.arange(max_len, dtype=jnp.float32)
    angles = jnp.outer(t, freqs)
    return jnp.cos(angles), jnp.sin(angles)


def _apply_rope(x, cos, sin):
    B, H, T, D = x.shape
    xr = x[..., 0::2]
    xi = x[..., 1::2]
    c = cos[None, :H, None, :]
    s = sin[None, :H, None, :]
    o_r = xr * c - xi * s
    o_i = xr * s + xi * c
    return jnp.stack([o_r, o_i], axis=-1).reshape(B, H, T, D)


def _transformer_block(x, p, cos, sin, n_heads):
    B, T, D = x.shape
    hd = D // n_heads
    h = _layer_norm(x, p['ln1_g'], p['ln1_b'])
    q = (h @ p['wq'].T + p['bq']).reshape(B, T, n_heads, hd).transpose(0, 2, 1, 3)
    k = (h @ p['wk'].T + p['bk']).reshape(B, T, n_heads, hd).transpose(0, 2, 1, 3)
    v = (h @ p['wv'].T + p['bv']).reshape(B, T, n_heads, hd).transpose(0, 2, 1, 3)
    q = _apply_rope(q, cos, sin)
    k = _apply_rope(k, cos, sin)
    scores = q @ k.transpose(0, 1, 3, 2) / np.sqrt(hd).astype(np.float32)
    mask = jnp.triu(jnp.ones((T, T), dtype=bool), k=1)
    scores = jnp.where(mask[None, None], -1e9, scores)
    attn = jax.nn.softmax(scores, axis=-1)
    o = (attn @ v).transpose(0, 2, 1, 3).reshape(B, T, D)
    x = x + o @ p['wo'].T + p['bo']
    h2 = _layer_norm(x, p['ln2_g'], p['ln2_b'])
    ffn = (_silu(h2 @ p['wg'].T + p['bg']) * (h2 @ p['wu'].T + p['bu'])) @ p['wd'].T + p['bd']
    return x + ffn


def _res_unit(x, p, pre):
    r = _conv1d(x, p[pre + '1w'], p[pre + '1b'], 1, 1, 1)
    r = _silu(r)
    r = _conv1d(r, p[pre + '2w'], p[pre + '2b'], 1, 3, 3)
    r = _silu(r)
    r = _conv1d(r, p[pre + '3w'], p[pre + '3b'], 1, 9, 9)
    return x + r


def kernel(audio, params):
    x = audio[:, None, :]
    x = _conv1d(x, params['enc_in_w'], params['enc_in_b'], 1, 3)
    for blk, s in zip(params['enc_blocks'], ENC_STRIDES):
        x = _conv1d(x, blk['dw'], blk['db'], s, s // 2)
        x = _res_unit(x, blk, 'r')
    x = _conv1d(x, params['enc_out_w'], params['enc_out_b'], 1, 0)
    lat = x.transpose(0, 2, 1)
    cos, sin = _rope_tables(ENC_DIM // 8, MAX_SEQ_LEN)
    for p in params['pre_tf']:
        lat = _transformer_block(lat, p, cos, sin, N_HEADS)
    zq = _rvq_quantize(lat, params['codebooks'])
    for p in params['post_tf']:
        zq = _transformer_block(zq, p, cos, sin, N_HEADS)
    y = zq.transpose(0, 2, 1)
    y = _conv1d(y, params['dec_in_w'], params['dec_in_b'], 1, 3)
    for blk, s in zip(params['dec_blocks'], DEC_STRIDES):
        y = _conv_transpose1d(y, blk['uw'], blk['ub'], s, s // 2)
        y = _res_unit(y, blk, 'r')
    y = _silu(y)
    y = _conv1d(y, params['dec_out_w'], params['dec_out_b'], 1, 3)
    y = jnp.tanh(y)
    return y[:, 0, :]
